# baseline (speedup 1.0000x reference)
import jax
import jax.numpy as jnp
from jax import lax
from jax.experimental import pallas as pl
from jax.experimental.pallas import tpu as pltpu
from jax.experimental.pallas import tpu_sc as plsc

B = 128
V = 100000
N_FEAT = 25
TOPK = 10

CHUNK = 20000
N_CHUNKS = V // CHUNK
GROUP = 25
N_GROUPS = CHUNK // (16 * GROUP)
ROWS_PER_WORKER = 4

NEG_INF = float("-inf")


def _merge16(best_asc, v):
    v_desc, _ = plsc.sort_key_val(v, v, descending=True)
    h = jnp.maximum(best_asc, v_desc)
    h_asc, _ = plsc.sort_key_val(h, h, descending=False)
    return h_asc


def _sc_body(item_hbm, feat_hbm, out1_hbm, out2_hbm,
             buf, fbuf, ov1, ov2):
    nc = 2
    wid = lax.axis_index("s") * nc + lax.axis_index("c")

    for r in range(ROWS_PER_WORKER):
        row = wid * ROWS_PER_WORKER + r
        best = jnp.full((16,), NEG_INF, jnp.float32)
        t = jnp.float32(NEG_INF)

        for c in range(N_CHUNKS):
            pltpu.sync_copy(
                item_hbm.at[pl.ds(row * jnp.int32(V) + jnp.int32(c * CHUNK),
                                  CHUNK)],
                buf)

            def gbody(g, carry):
                best, t = carry
                base = g * jnp.int32(16 * GROUP)
                acc = buf[pl.ds(base, 16)]
                for j in range(1, GROUP):
                    acc = jnp.maximum(acc,
                                      buf[pl.ds(base + jnp.int32(16 * j), 16)])
                gmax = jnp.max(acc)

                def rescan(carry2):
                    def jbody(j, carry3):
                        best3, t3 = carry3
                        v = buf[pl.ds(base + j * jnp.int32(16), 16)]
                        vmax = jnp.max(v)

                        def merge(c4):
                            b4, _ = c4
                            nb = _merge16(b4, v)
                            return nb, jnp.min(nb)

                        return lax.cond(vmax > t3, merge, lambda c4: c4,
                                        (best3, t3))

                    return lax.fori_loop(jnp.int32(0), jnp.int32(GROUP),
                                         jbody, carry2)

                return lax.cond(gmax > t, rescan, lambda c2: c2, (best, t))

            best, t = lax.fori_loop(jnp.int32(0), jnp.int32(N_GROUPS),
                                    gbody, (best, t))

        bdesc, _ = plsc.sort_key_val(best, best, descending=True)
        lane = lax.iota(jnp.int32, 16)
        top = jnp.where(lane < TOPK, bdesc, NEG_INF)

        pltpu.sync_copy(feat_hbm.at[pl.ds(row * jnp.int32(32), 32)], fbuf)
        f0 = fbuf[pl.ds(0, 16)]
        f1 = fbuf[pl.ds(16, 16)]

        m = jnp.maximum(jnp.maximum(jnp.max(f0), jnp.max(f1)), jnp.max(top))
        e0 = jnp.exp(f0 - m)
        e1 = jnp.exp(f1 - m)
        et = jnp.exp(top - m)
        es = e0 + e1 + et
        for k in (1, 2, 4, 8):
            es = es + es[jnp.bitwise_xor(lane, k)]
        inv = jnp.float32(1.0) / es
        ov1[pl.ds(0, 16)] = e0 * inv
        ov1[pl.ds(16, 16)] = e1 * inv
        ov2[...] = et * inv
        pltpu.sync_copy(ov1, out1_hbm.at[pl.ds(row * jnp.int32(32), 32)])
        pltpu.sync_copy(ov2, out2_hbm.at[pl.ds(row * jnp.int32(16), 16)])


def kernel(item_scores, feat_scores, cand_item):
    del cand_item
    feat = jnp.pad(feat_scores.astype(jnp.float32),
                   ((0, 0), (0, 32 - N_FEAT)), constant_values=-jnp.inf)

    mesh = plsc.VectorSubcoreMesh(core_axis_name="c", subcore_axis_name="s")
    run = pl.kernel(
        _sc_body,
        mesh=mesh,
        out_type=[
            jax.ShapeDtypeStruct((B * 32,), jnp.float32),
            jax.ShapeDtypeStruct((B * 16,), jnp.float32),
        ],
        scratch_types=[
            pltpu.VMEM((CHUNK,), jnp.float32),
            pltpu.VMEM((32,), jnp.float32),
            pltpu.VMEM((32,), jnp.float32),
            pltpu.VMEM((16,), jnp.float32),
        ],
        compiler_params=pltpu.CompilerParams(needs_layout_passes=False),
    )
    out1, out2 = run(item_scores.reshape(-1), feat.reshape(-1))
    out1 = out1.reshape(B, 32)
    out2 = out2.reshape(B, 16)
    return jnp.concatenate([out1[:, :N_FEAT], out2[:, :TOPK]], axis=1)


# speedup vs baseline: 1.5763x; 1.5763x over previous
import jax
import jax.numpy as jnp
from jax import lax
from jax.experimental import pallas as pl
from jax.experimental.pallas import tpu as pltpu
from jax.experimental.pallas import tpu_sc as plsc

B = 128
V = 100000
N_FEAT = 25
TOPK = 10

CHUNK = 20000
N_CHUNKS = V // CHUNK
GROUP = 25
N_GROUPS = CHUNK // (16 * GROUP)
ROWS_PER_WORKER = 4
CAND_CAP = 16 * GROUP

NEG_INF = float("-inf")


def _merge16(best_asc, v):
    v_desc, _ = plsc.sort_key_val(v, v, descending=True)
    h = jnp.maximum(best_asc, v_desc)
    h_asc, _ = plsc.sort_key_val(h, h, descending=False)
    return h_asc


def _tree_max(vals):
    while len(vals) > 1:
        nxt = [jnp.maximum(vals[i], vals[i + 1])
               for i in range(0, len(vals) - 1, 2)]
        if len(vals) % 2:
            nxt.append(vals[-1])
        vals = nxt
    return vals[0]


def _sc_body(item_hbm, feat_hbm, out1_hbm, out2_hbm,
             buf0, buf1, cand, fbuf, ov1, ov2, sem0, sem1):
    nc = 2
    wid = lax.axis_index("s") * nc + lax.axis_index("c")
    lane = lax.iota(jnp.int32, 16)
    six = jnp.full((16,), 6, jnp.int32)
    bufs = (buf0, buf1)
    sems = (sem0, sem1)

    n_work = ROWS_PER_WORKER * N_CHUNKS

    def chunk_slice(k):
        row = k // N_CHUNKS
        c = k % N_CHUNKS
        off = (wid * ROWS_PER_WORKER + row) * jnp.int32(V) \
            + jnp.int32(c * CHUNK)
        return item_hbm.at[pl.ds(off, CHUNK)]

    cp = pltpu.make_async_copy(chunk_slice(0), bufs[0], sems[0])
    cp.start()

    best = None
    t_splat = None
    for k in range(n_work):
        row = k // N_CHUNKS
        c = k % N_CHUNKS
        buf = bufs[k % 2]
        if c == 0:
            best = jnp.full((16,), NEG_INF, jnp.float32)
            t_splat = jnp.full((16,), NEG_INF, jnp.float32)
        if k + 1 < n_work:
            nxt = pltpu.make_async_copy(chunk_slice(k + 1),
                                        bufs[(k + 1) % 2],
                                        sems[(k + 1) % 2])
            nxt.start()
        pltpu.make_async_copy(chunk_slice(k), buf, sems[k % 2]).wait()

        def gbody(g, carry, buf=buf):
            best, t_splat = carry
            base = g * jnp.int32(16 * GROUP)
            vs = [buf[pl.ds(base + jnp.int32(16 * j), 16)]
                  for j in range(GROUP)]
            acc = _tree_max(vs)
            cnt = plsc.all_reduce_population_count(acc > t_splat)
            pred = cnt[0] > 0

            def rescan(carry2, buf=buf, base=base):
                best2, t2 = carry2

                def collect(j, basev):
                    v = buf[pl.ds(base + j * jnp.int32(16), 16)]
                    msk = v > t2
                    cum = plsc.cumsum(msk.astype(jnp.int32))
                    n = plsc.all_reduce_population_count(msk)
                    idx = jnp.maximum(basev + cum - 1, 0)
                    plsc.store_scatter(cand, [idx], v, mask=msk)
                    return basev + n

                basev = lax.fori_loop(jnp.int32(0), jnp.int32(GROUP), collect,
                                      jnp.zeros((16,), jnp.int32))
                ncand = basev[0]

                def wcond(carry3):
                    i, _ = carry3
                    return i * 16 < ncand

                def wbody(carry3):
                    i, b = carry3
                    w = cand[pl.ds(i * jnp.int32(16), 16)]
                    valid = (i * 16 + lane) < ncand
                    w = jnp.where(valid, w, NEG_INF)
                    return i + 1, _merge16(b, w)

                _, best2 = lax.while_loop(wcond, wbody,
                                          (jnp.int32(0), best2))
                return best2, best2[six]

            return lax.cond(pred, rescan, lambda c2: c2, (best, t_splat))

        best, t_splat = lax.fori_loop(jnp.int32(0), jnp.int32(N_GROUPS),
                                      gbody, (best, t_splat))

        if c == N_CHUNKS - 1:
            rowg = wid * ROWS_PER_WORKER + row
            bdesc, _ = plsc.sort_key_val(best, best, descending=True)
            top = jnp.where(lane < TOPK, bdesc, NEG_INF)

            pltpu.sync_copy(feat_hbm.at[pl.ds(rowg * jnp.int32(32), 32)],
                            fbuf)
            f0 = fbuf[pl.ds(0, 16)]
            f1 = fbuf[pl.ds(16, 16)]

            mx = jnp.maximum(jnp.maximum(f0, f1), top)
            for s in (1, 2, 4, 8):
                mx = jnp.maximum(mx, mx[jnp.bitwise_xor(lane, s)])
            e0 = jnp.exp(f0 - mx)
            e1 = jnp.exp(f1 - mx)
            et = jnp.exp(top - mx)
            es = e0 + e1 + et
            for s in (1, 2, 4, 8):
                es = es + es[jnp.bitwise_xor(lane, s)]
            inv = jnp.float32(1.0) / es
            ov1[pl.ds(0, 16)] = e0 * inv
            ov1[pl.ds(16, 16)] = e1 * inv
            ov2[...] = et * inv
            pltpu.sync_copy(ov1, out1_hbm.at[pl.ds(rowg * jnp.int32(32), 32)])
            pltpu.sync_copy(ov2, out2_hbm.at[pl.ds(rowg * jnp.int32(16), 16)])


def kernel(item_scores, feat_scores, cand_item):
    del cand_item
    feat = jnp.pad(feat_scores.astype(jnp.float32),
                   ((0, 0), (0, 32 - N_FEAT)), constant_values=-jnp.inf)

    mesh = plsc.VectorSubcoreMesh(core_axis_name="c", subcore_axis_name="s")
    run = pl.kernel(
        _sc_body,
        mesh=mesh,
        out_type=[
            jax.ShapeDtypeStruct((B * 32,), jnp.float32),
            jax.ShapeDtypeStruct((B * 16,), jnp.float32),
        ],
        scratch_types=[
            pltpu.VMEM((CHUNK,), jnp.float32),
            pltpu.VMEM((CHUNK,), jnp.float32),
            pltpu.VMEM((CAND_CAP,), jnp.float32),
            pltpu.VMEM((32,), jnp.float32),
            pltpu.VMEM((32,), jnp.float32),
            pltpu.VMEM((16,), jnp.float32),
            pltpu.SemaphoreType.DMA,
            pltpu.SemaphoreType.DMA,
        ],
        compiler_params=pltpu.CompilerParams(needs_layout_passes=False),
    )
    out1, out2 = run(item_scores.reshape(-1), feat.reshape(-1))
    out1 = out1.reshape(B, 32)
    out2 = out2.reshape(B, 16)
    return jnp.concatenate([out1[:, :N_FEAT], out2[:, :TOPK]], axis=1)


# speedup vs baseline: 2.5882x; 1.6420x over previous
import jax
import jax.numpy as jnp
from jax import lax
from jax.experimental import pallas as pl
from jax.experimental.pallas import tpu as pltpu
from jax.experimental.pallas import tpu_sc as plsc

B = 128
V = 100000
N_FEAT = 25
TOPK = 10

N_OCT = 16
CW = 4992
N_CHUNKS = 10
HALF = CW * N_CHUNKS
REM_OFF = 2 * HALF
REM = V - REM_OFF
GROUP = 13
N_GROUPS = CW // (16 * GROUP)
CAND_CAP = 256

NEG_INF = float("-inf")


def _merge16(best_asc, v):
    v_desc, _ = plsc.sort_key_val(v, v, descending=True)
    h = jnp.maximum(best_asc, v_desc)
    h_asc, _ = plsc.sort_key_val(h, h, descending=False)
    return h_asc


def _tree_max(vals):
    while len(vals) > 1:
        nxt = [jnp.maximum(vals[i], vals[i + 1])
               for i in range(0, len(vals) - 1, 2)]
        if len(vals) % 2:
            nxt.append(vals[-1])
        vals = nxt
    return vals[0]


def _sc_body(item_hbm, feat_hbm, out1_hbm, out2_hbm,
             buf0, buf1, rbuf, cand, fbuf, bbuf, pbuf, o1buf, o2buf,
             shared, sem0, sem1):
    s_idx = lax.axis_index("s")
    c_idx = lax.axis_index("c")
    oct_i = c_idx * jnp.int32(8) + lax.div(s_idx, jnp.int32(2))
    h = lax.rem(s_idx, jnp.int32(2))
    lane = lax.iota(jnp.int32, 16)
    six = jnp.full((16,), 6, jnp.int32)
    col0 = h * jnp.int32(HALF)

    def chunk_slice(c):
        return item_hbm.at[oct_i, :, pl.ds(col0 + c * jnp.int32(CW), CW)]

    def scan_group(vs, best, t_splat):
        acc = _tree_max(vs)
        cnt = plsc.all_reduce_population_count(acc > t_splat)
        pred = cnt[0] > 0

        def rescan(carry2):
            best2, t2 = carry2
            basev = jnp.zeros((16,), jnp.int32)
            for v in vs:
                msk = v > t2
                cum = plsc.cumsum(msk.astype(jnp.int32))
                n = plsc.all_reduce_population_count(msk)
                idx = jnp.maximum(basev + cum - 1, 0)
                plsc.store_scatter(cand, [idx], v, mask=msk)
                basev = basev + n
            ncand = basev[0]

            def wcond(carry3):
                i, _ = carry3
                return i * 16 < ncand

            def wbody(carry3):
                i, b = carry3
                w = cand[pl.ds(i * jnp.int32(16), 16)]
                valid = (i * 16 + lane) < ncand
                w = jnp.where(valid, w, NEG_INF)
                return i + 1, _merge16(b, w)

            _, best2 = lax.while_loop(wcond, wbody, (jnp.int32(0), best2))
            return best2, best2[six]

        return lax.cond(pred, rescan, lambda c2: c2, (best, t_splat))

    def process(buf, carry):
        def gbody(g, carry):
            bs, ts = list(carry[0]), list(carry[1])
            base = g * jnp.int32(16 * GROUP)
            for r in range(8):
                vs = [buf[r, pl.ds(base + jnp.int32(16 * j), 16)]
                      for j in range(GROUP)]
                bs[r], ts[r] = scan_group(vs, bs[r], ts[r])
            return tuple(bs), tuple(ts)

        return lax.fori_loop(jnp.int32(0), jnp.int32(N_GROUPS), gbody, carry)

    pltpu.make_async_copy(chunk_slice(jnp.int32(0)), buf0, sem0).start()
    pltpu.make_async_copy(chunk_slice(jnp.int32(1)), buf1, sem1).start()
    init = (tuple(jnp.full((16,), NEG_INF, jnp.float32) for _ in range(8)),
            tuple(jnp.full((16,), NEG_INF, jnp.float32) for _ in range(8)))

    def super_body(si, carry):
        c0 = si * jnp.int32(2)
        pltpu.make_async_copy(chunk_slice(c0), buf0, sem0).wait()
        carry = process(buf0, carry)
        nxt0 = jnp.minimum(c0 + 2, jnp.int32(8))
        pltpu.make_async_copy(chunk_slice(nxt0), buf0, sem0).start()
        pltpu.make_async_copy(chunk_slice(c0 + 1), buf1, sem1).wait()
        carry = process(buf1, carry)
        nxt1 = jnp.minimum(c0 + 3, jnp.int32(9))
        pltpu.make_async_copy(chunk_slice(nxt1), buf1, sem1).start()
        return carry

    carry = lax.fori_loop(jnp.int32(0), jnp.int32(5), super_body, init)
    pltpu.make_async_copy(chunk_slice(jnp.int32(8)), buf0, sem0).wait()
    pltpu.make_async_copy(chunk_slice(jnp.int32(9)), buf1, sem1).wait()

    pltpu.sync_copy(item_hbm.at[oct_i, :, pl.ds(jnp.int32(REM_OFF), REM)],
                    rbuf)
    bs, ts = list(carry[0]), list(carry[1])
    rbase = h * jnp.int32(REM // 2)
    for r in range(8):
        vs = [rbuf[r, pl.ds(rbase + jnp.int32(16 * j), 16)]
              for j in range(REM // 32)]
        bs[r], ts[r] = scan_group(vs, bs[r], ts[r])

    for r in range(8):
        bbuf[r, :] = bs[r]
    pltpu.sync_copy(bbuf, shared.at[s_idx])
    plsc.subcore_barrier()

    @pl.when(h == 0)
    def _finalize():
        pltpu.sync_copy(shared.at[s_idx + 1], pbuf)
        pltpu.sync_copy(feat_hbm.at[oct_i], fbuf)
        for r in range(8):
            merged = _merge16(bs[r], pbuf[r, :])
            bdesc, _ = plsc.sort_key_val(merged, merged, descending=True)
            top = jnp.where(lane < TOPK, bdesc, NEG_INF)
            f0 = fbuf[r, pl.ds(0, 16)]
            f1 = fbuf[r, pl.ds(16, 16)]
            mx = jnp.maximum(jnp.maximum(f0, f1), top)
            for s in (1, 2, 4, 8):
                mx = jnp.maximum(mx, mx[jnp.bitwise_xor(lane, s)])
            e0 = jnp.exp(f0 - mx)
            e1 = jnp.exp(f1 - mx)
            et = jnp.exp(top - mx)
            es = e0 + e1 + et
            for s in (1, 2, 4, 8):
                es = es + es[jnp.bitwise_xor(lane, s)]
            inv = jnp.float32(1.0) / es
            o1buf[r, pl.ds(0, 16)] = e0 * inv
            o1buf[r, pl.ds(16, 16)] = e1 * inv
            o2buf[r, :] = et * inv
        pltpu.sync_copy(o1buf, out1_hbm.at[oct_i])
        pltpu.sync_copy(o2buf, out2_hbm.at[oct_i])


def kernel(item_scores, feat_scores, cand_item):
    del cand_item
    feat = jnp.pad(feat_scores.astype(jnp.float32),
                   ((0, 0), (0, 32 - N_FEAT)), constant_values=-jnp.inf)

    mesh = plsc.VectorSubcoreMesh(core_axis_name="c", subcore_axis_name="s")
    run = pl.kernel(
        _sc_body,
        mesh=mesh,
        out_type=[
            jax.ShapeDtypeStruct((N_OCT, 8, 32), jnp.float32),
            jax.ShapeDtypeStruct((N_OCT, 8, 16), jnp.float32),
        ],
        scratch_types=[
            pltpu.VMEM((8, CW), jnp.float32),
            pltpu.VMEM((8, CW), jnp.float32),
            pltpu.VMEM((8, REM), jnp.float32),
            pltpu.VMEM((CAND_CAP,), jnp.float32),
            pltpu.VMEM((8, 32), jnp.float32),
            pltpu.VMEM((8, 16), jnp.float32),
            pltpu.VMEM((8, 16), jnp.float32),
            pltpu.VMEM((8, 32), jnp.float32),
            pltpu.VMEM((8, 16), jnp.float32),
            pltpu.VMEM_SHARED((16, 8, 16), jnp.float32),
            pltpu.SemaphoreType.DMA,
            pltpu.SemaphoreType.DMA,
        ],
        compiler_params=pltpu.CompilerParams(needs_layout_passes=False),
    )
    out1, out2 = run(item_scores.reshape(N_OCT, 8, V),
                     feat.reshape(N_OCT, 8, 32))
    out1 = out1.reshape(B, 32)
    out2 = out2.reshape(B, 16)
    return jnp.concatenate([out1[:, :N_FEAT], out2[:, :TOPK]], axis=1)
